# baseline (speedup 1.0000x reference)
import jax
import jax.numpy as jnp
from jax import lax
from jax.experimental import pallas as pl
from jax.experimental.pallas import tpu as pltpu

B = 16384
L = 200
CBLK = 16384
GRID = B // CBLK
SUB = CBLK // 128


def _tc_kernel(node_ref, items_ref, out_ref):
    items = items_ref[...]
    l_iota = lax.broadcasted_iota(jnp.int32, (L, 128), 0)
    for s in range(SUB):
        node_s = node_ref[s:s + 1, :]
        pick = l_iota == node_s - 1
        sub = items[:, s * 128:(s + 1) * 128]
        sel = jnp.where(pick, sub, jnp.int32(0))
        tot = jnp.sum(sel, axis=0, keepdims=True)
        out_ref[s:s + 1, :] = jnp.where(node_s != 0, tot, jnp.int32(-1))


@jax.jit
def _decode(node2d, items_t):
    return pl.pallas_call(
        _tc_kernel,
        grid=(GRID,),
        in_specs=[
            pl.BlockSpec((SUB, 128), lambda i: (i, 0)),
            pl.BlockSpec((L, CBLK), lambda i: (0, i)),
        ],
        out_specs=pl.BlockSpec((SUB, 128), lambda i: (i, 0)),
        out_shape=jax.ShapeDtypeStruct((B // 128, 128), jnp.int32),
        compiler_params=pltpu.CompilerParams(
            dimension_semantics=("arbitrary",),
        ),
    )(node2d, items_t)


def kernel(x_dummy, current_node, item_ids):
    node2d = jnp.reshape(current_node.astype(jnp.int32), (B // 128, 128))
    items_t = jnp.transpose(item_ids.astype(jnp.int32))
    out = _decode(node2d, items_t)
    return jnp.reshape(out, (B,)).astype(item_ids.dtype)
